# Initial kernel scaffold; baseline (speedup 1.0000x reference)
import jax
import jax.numpy as jnp
from jax.experimental import pallas as pl


def kernel(content_feats, style_feats, content_batch_indices, style_batch_indices):
    raise NotImplementedError("write your pallas kernel here")


# speedup vs baseline: 7.3888x; 7.3888x over previous
import numpy as np
import jax
import jax.numpy as jnp
from jax.experimental import pallas as pl
from jax.experimental.pallas import tpu as pltpu

_B = 16
_ALPHA = 0.1
_EPS = 1e-8


def _ema_weight_matrix():
    w = np.zeros((_B, _B), dtype=np.float32)
    w[0, 0] = 1.0
    for b in range(1, _B):
        w[b] = w[b - 1] * (1.0 - _ALPHA)
        w[b, b] = _ALPHA
    return w


_W = _ema_weight_matrix()


def _stats_kernel(cf, sf, ci, si, c_sum, c_ssq, c_cnt, s_sum, s_ssq, s_cnt):
    r = cf.shape[0]
    seg = jax.lax.broadcasted_iota(jnp.int32, (r, _B), 1)
    oh_c = (ci[...] == seg).astype(jnp.float32)
    oh_s = (si[...] == seg).astype(jnp.float32)
    x = cf[...]
    y = sf[...]
    dims = (((0,), (0,)), ((), ()))
    ps = jax.lax.dot_general(oh_c, x, dims, preferred_element_type=jnp.float32)
    pq = jax.lax.dot_general(oh_c, x * x, dims, preferred_element_type=jnp.float32)
    pc = jnp.broadcast_to(jnp.sum(oh_c, axis=0)[:, None], (_B, cf.shape[1]))
    ts = jax.lax.dot_general(oh_s, y, dims, preferred_element_type=jnp.float32)
    tq = jax.lax.dot_general(oh_s, y * y, dims, preferred_element_type=jnp.float32)
    tc = jnp.broadcast_to(jnp.sum(oh_s, axis=0)[:, None], (_B, sf.shape[1]))

    @pl.when(pl.program_id(0) == 0)
    def _():
        c_sum[...] = ps
        c_ssq[...] = pq
        c_cnt[...] = pc
        s_sum[...] = ts
        s_ssq[...] = tq
        s_cnt[...] = tc

    @pl.when(pl.program_id(0) != 0)
    def _():
        c_sum[...] += ps
        c_ssq[...] += pq
        c_cnt[...] += pc
        s_sum[...] += ts
        s_ssq[...] += tq
        s_cnt[...] += tc


def _finalize_kernel(c_sum, c_ssq, c_cnt, s_sum, s_ssq, s_cnt, w, scale, offset):
    ccnt = c_cnt[...]
    cmean = c_sum[...] / ccnt
    cvar = (c_ssq[...] - ccnt * cmean * cmean) / (ccnt - 1.0)
    cstd = jnp.sqrt(jnp.maximum(cvar, 0.0)) + _EPS
    scnt = s_cnt[...]
    smean = s_sum[...] / scnt
    svar = (s_ssq[...] - scnt * smean * smean) / (scnt - 1.0)
    sstd = jnp.sqrt(jnp.maximum(svar, 0.0)) + _EPS
    dims = (((1,), (0,)), ((), ()))
    gmean = jax.lax.dot_general(w[...], smean, dims, preferred_element_type=jnp.float32)
    gstd = jax.lax.dot_general(w[...], sstd, dims, preferred_element_type=jnp.float32)
    sc = gstd / cstd
    scale[...] = sc
    offset[...] = gmean - sc * cmean


def _norm_kernel(cf, ci, scale, offset, out):
    r = cf.shape[0]
    seg = jax.lax.broadcasted_iota(jnp.int32, (r, _B), 1)
    oh = (ci[...] == seg).astype(jnp.float32)
    dims = (((1,), (0,)), ((), ()))
    rs = jax.lax.dot_general(oh, scale[...], dims, preferred_element_type=jnp.float32)
    ro = jax.lax.dot_general(oh, offset[...], dims, preferred_element_type=jnp.float32)
    out[...] = cf[...] * rs + ro


def _pick_block(n):
    for r in (4000, 2000, 1600, 1000, 800, 500, 400, 200, 100, 8):
        if n % r == 0:
            return r
    return None


def kernel(content_feats, style_feats, content_batch_indices, style_batch_indices):
    n, c = content_feats.shape
    ns = style_feats.shape[0]
    assert ns == n, "kernel assumes matching content/style row counts"
    r = _pick_block(n)
    nb = n // r
    ci = content_batch_indices.reshape(n, 1)
    si = style_batch_indices.reshape(n, 1)

    stat_shape = jax.ShapeDtypeStruct((_B, c), jnp.float32)
    stat_spec = pl.BlockSpec((_B, c), lambda i: (0, 0))
    stats = pl.pallas_call(
        _stats_kernel,
        grid=(nb,),
        in_specs=[
            pl.BlockSpec((r, c), lambda i: (i, 0)),
            pl.BlockSpec((r, c), lambda i: (i, 0)),
            pl.BlockSpec((r, 1), lambda i: (i, 0)),
            pl.BlockSpec((r, 1), lambda i: (i, 0)),
        ],
        out_specs=[stat_spec] * 6,
        out_shape=[stat_shape] * 6,
    )(content_feats, style_feats, ci, si)
    c_sum, c_ssq, c_cnt, s_sum, s_ssq, s_cnt = stats

    w = jnp.asarray(_W)
    scale, offset = pl.pallas_call(
        _finalize_kernel,
        out_shape=[stat_shape, stat_shape],
    )(c_sum, c_ssq, c_cnt, s_sum, s_ssq, s_cnt, w)

    out = pl.pallas_call(
        _norm_kernel,
        grid=(nb,),
        in_specs=[
            pl.BlockSpec((r, c), lambda i: (i, 0)),
            pl.BlockSpec((r, 1), lambda i: (i, 0)),
            pl.BlockSpec((_B, c), lambda i: (0, 0)),
            pl.BlockSpec((_B, c), lambda i: (0, 0)),
        ],
        out_specs=pl.BlockSpec((r, c), lambda i: (i, 0)),
        out_shape=jax.ShapeDtypeStruct((n, c), jnp.float32),
    )(content_feats, ci, scale, offset)
    return out
